# baseline (speedup 1.0000x reference)
import functools

import jax
import jax.numpy as jnp
from jax import lax
from jax.experimental import pallas as pl
from jax.experimental.pallas import tpu as pltpu
from jax.experimental.pallas import tpu_sc as plsc

_B, _L = 16384, 50
_D = 64
_N = _B * _L

_info = plsc.get_sparse_core_info()
_NC, _NS = _info.num_cores, _info.num_subcores
_NW = _NC * _NS
_PER_W = _N // _NW
_CHUNK = 512
_NCHUNKS = _PER_W // _CHUNK
_NBUF = 2


def _make_gather():
    mesh = plsc.VectorSubcoreMesh(core_axis_name="c", subcore_axis_name="s")

    @functools.partial(
        pl.kernel,
        mesh=mesh,
        out_type=jax.ShapeDtypeStruct((_N, _D), jnp.float16),
        compiler_params=pltpu.CompilerParams(use_tc_tiling_on_sc=False),
        scratch_types=[
            pltpu.VMEM((_PER_W,), jnp.int32),
            pltpu.VMEM((_NBUF, _CHUNK, _D), jnp.float16),
            pltpu.SemaphoreType.DMA((_NBUF,)),
            pltpu.SemaphoreType.DMA((_NBUF,)),
        ],
    )
    def gather_kernel(idx_hbm, table_hbm, out_hbm, idx_v, bufs, gsem, ssem):
        wid = lax.axis_index("s") * _NC + lax.axis_index("c")
        w_base = wid * _PER_W
        pltpu.sync_copy(idx_hbm.at[pl.ds(w_base, _PER_W)], idx_v)

        def start_gather(b, c):
            pltpu.async_copy(
                table_hbm.at[idx_v.at[pl.ds(c * _CHUNK, _CHUNK)]],
                bufs.at[b],
                gsem.at[b],
            )

        def wait_gather(b):
            pltpu.make_async_copy(
                table_hbm.at[idx_v.at[pl.ds(0, _CHUNK)]],
                bufs.at[b],
                gsem.at[b],
            ).wait()

        def start_store(b, c):
            pltpu.async_copy(
                bufs.at[b],
                out_hbm.at[pl.ds(w_base + c * _CHUNK, _CHUNK)],
                ssem.at[b],
            )

        def wait_store(b):
            pltpu.make_async_copy(
                bufs.at[b],
                out_hbm.at[pl.ds(w_base, _CHUNK)],
                ssem.at[b],
            ).wait()

        start_gather(0, 0)

        def body(io, carry):
            for u in range(_NBUF):
                c = io * _NBUF + u
                b = u
                nb = (u + 1) % _NBUF
                wait_gather(b)
                start_store(b, c)

                @pl.when(c + 1 < _NCHUNKS)
                def _():
                    @pl.when(c + 1 >= _NBUF)
                    def _():
                        wait_store(nb)

                    start_gather(nb, c + 1)

            return carry

        lax.fori_loop(0, _NCHUNKS // _NBUF, body, 0)
        for b in range(_NBUF):
            wait_store(b)

    return gather_kernel


_gather = _make_gather()


def kernel(token_ids, weight):
    idx = token_ids.reshape(_N).astype(jnp.int32)
    table_h = weight.astype(jnp.float16)
    staged = _gather(idx, table_h)
    return staged.astype(jnp.float32).reshape(_B, _L, _D)


# speedup vs baseline: 1.8032x; 1.8032x over previous
import functools

import jax
import jax.numpy as jnp
from jax import lax
from jax.experimental import pallas as pl
from jax.experimental.pallas import tpu as pltpu
from jax.experimental.pallas import tpu_sc as plsc

_B, _L = 16384, 50
_D = 64
_N = _B * _L

_info = plsc.get_sparse_core_info()
_NC, _NS = _info.num_cores, _info.num_subcores
_NW = _NC * _NS
_PER_W = _N // _NW
_CHUNK = 640
_NCHUNKS = _PER_W // _CHUNK
_NBUF = 2


def _make_gather():
    mesh = plsc.VectorSubcoreMesh(core_axis_name="c", subcore_axis_name="s")

    @functools.partial(
        pl.kernel,
        mesh=mesh,
        out_type=jax.ShapeDtypeStruct((_N, _D), jnp.float32),
        compiler_params=pltpu.CompilerParams(use_tc_tiling_on_sc=False),
        scratch_types=[
            pltpu.VMEM((_PER_W,), jnp.int32),
            pltpu.VMEM((_NBUF, _CHUNK, _D), jnp.float32),
            pltpu.SemaphoreType.DMA((_NBUF,)),
            pltpu.SemaphoreType.DMA((_NBUF,)),
        ],
    )
    def gather_kernel(idx_hbm, table_hbm, out_hbm, idx_v, bufs, gsem, ssem):
        wid = lax.axis_index("s") * _NC + lax.axis_index("c")
        w_base = wid * _PER_W
        pltpu.sync_copy(idx_hbm.at[pl.ds(w_base, _PER_W)], idx_v)

        def start_gather(b, c):
            pltpu.async_copy(
                table_hbm.at[idx_v.at[pl.ds(c * _CHUNK, _CHUNK)]],
                bufs.at[b],
                gsem.at[b],
            )

        def wait_gather(b):
            pltpu.make_async_copy(
                table_hbm.at[idx_v.at[pl.ds(0, _CHUNK)]],
                bufs.at[b],
                gsem.at[b],
            ).wait()

        def start_store(b, c):
            pltpu.async_copy(
                bufs.at[b],
                out_hbm.at[pl.ds(w_base + c * _CHUNK, _CHUNK)],
                ssem.at[b],
            )

        def wait_store(b):
            pltpu.make_async_copy(
                bufs.at[b],
                out_hbm.at[pl.ds(w_base, _CHUNK)],
                ssem.at[b],
            ).wait()

        start_gather(0, 0)

        def body(io, carry):
            for u in range(_NBUF):
                c = io * _NBUF + u
                b = u
                nb = (u + 1) % _NBUF
                wait_gather(b)
                start_store(b, c)

                @pl.when(c + 1 < _NCHUNKS)
                def _():
                    @pl.when(c + 1 >= _NBUF)
                    def _():
                        wait_store(nb)

                    start_gather(nb, c + 1)

            return carry

        lax.fori_loop(0, _NCHUNKS // _NBUF, body, 0)
        for b in range(_NBUF):
            wait_store(b)

    return gather_kernel


_gather = _make_gather()


def kernel(token_ids, weight):
    idx = token_ids.reshape(_N).astype(jnp.int32)
    staged = _gather(idx, weight)
    return staged.reshape(_B, _L, _D)
